# baseline (speedup 1.0000x reference)
import functools

import jax
import jax.numpy as jnp
from jax import lax
from jax.experimental import pallas as pl
from jax.experimental.pallas import tpu as pltpu
from jax.experimental.pallas import tpu_sc as plsc

BATCH = 16384
EMB = 32

_info = plsc.get_sparse_core_info()
_NC, _NS = _info.num_cores, _info.num_subcores
_NW = _NC * _NS
_BPW = BATCH // _NW
_CH = 256


def _gather_one(idx_hbm, tab_hbm, out_hbm, base, idx_v, off_v, blk_v, row_v,
                sem):
    pltpu.sync_copy(idx_hbm.at[pl.ds(base, _BPW)], idx_v)

    def _split(i, _):
        v = idx_v[pl.ds(i * 16, 16)]
        off_v[pl.ds(i * 16, 16)] = lax.shift_left(
            jnp.bitwise_and(v, 3), 5)
        idx_v[pl.ds(i * 16, 16)] = lax.shift_right_logical(v, 2)
        return _
    lax.fori_loop(0, _BPW // 16, _split, 0, unroll=4)

    iota = lax.iota(jnp.int32, 16)
    for c in range(_BPW // _CH):
        pltpu.async_copy(tab_hbm.at[idx_v.at[pl.ds(c * _CH, _CH)]], blk_v,
                         sem).wait()

        def _select(g, _):
            rows16 = g * 16 + iota
            cols16 = off_v[pl.ds(c * _CH + g * 16, 16)]
            for j in range(EMB):
                vals = plsc.load_gather(blk_v, [rows16, cols16 + j])
                plsc.store_scatter(row_v, [rows16, jnp.full((16,), j,
                                                            jnp.int32)], vals)
            return _
        lax.fori_loop(0, _CH // 16, _select, 0)
        pltpu.sync_copy(row_v, out_hbm.at[pl.ds(base + c * _CH, _CH)])


def _gather_body(users_hbm, movies_hbm, utab_hbm, mtab_hbm,
                 ue_hbm, me_hbm,
                 idx_v, off_v, blk_v, row_v, sem):
    wid = lax.axis_index("s") * _NC + lax.axis_index("c")
    base = wid * _BPW
    _gather_one(users_hbm, utab_hbm, ue_hbm, base, idx_v, off_v, blk_v,
                row_v, sem)
    _gather_one(movies_hbm, mtab_hbm, me_hbm, base, idx_v, off_v, blk_v,
                row_v, sem)


@jax.jit
def _sc_gather(users, movies, utab128, mtab128):
    mesh = plsc.VectorSubcoreMesh(core_axis_name="c", subcore_axis_name="s")
    fn = functools.partial(
        pl.kernel,
        mesh=mesh,
        compiler_params=pltpu.CompilerParams(needs_layout_passes=False),
        out_type=[
            jax.ShapeDtypeStruct((BATCH, EMB), jnp.float32),
            jax.ShapeDtypeStruct((BATCH, EMB), jnp.float32),
        ],
        scratch_types=[
            pltpu.VMEM((_BPW,), jnp.int32),
            pltpu.VMEM((_BPW,), jnp.int32),
            pltpu.VMEM((_CH, 128), jnp.float32),
            pltpu.VMEM((_CH, EMB), jnp.float32),
            pltpu.SemaphoreType.DMA,
        ],
    )(_gather_body)
    return fn(users, movies, utab128, mtab128)


def _mlp_body(ue_ref, me_ref, w1u_ref, w1m_ref, b1_ref, w2_ref, b2_ref,
              w3_ref, b3_ref, out_ref):
    h = jnp.dot(ue_ref[...], w1u_ref[...], preferred_element_type=jnp.float32)
    h += jnp.dot(me_ref[...], w1m_ref[...], preferred_element_type=jnp.float32)
    h = jax.nn.relu(h + b1_ref[...])
    h = jnp.dot(h, w2_ref[...], preferred_element_type=jnp.float32)
    h = jax.nn.relu(h + b2_ref[...])
    out = jnp.dot(h, w3_ref[...], preferred_element_type=jnp.float32)
    out_ref[...] = out + b3_ref[...]


@jax.jit
def _tc_mlp(ue, me, w1u, w1m, b1, w2, b2, w3, b3):
    blk = 2048
    grid = (BATCH // blk,)
    rep = lambda shape: pl.BlockSpec(shape, lambda i: (0, 0))
    return pl.pallas_call(
        _mlp_body,
        grid=grid,
        in_specs=[
            pl.BlockSpec((blk, EMB), lambda i: (i, 0)),
            pl.BlockSpec((blk, EMB), lambda i: (i, 0)),
            rep(w1u.shape),
            rep(w1m.shape),
            rep(b1.shape),
            rep(w2.shape),
            rep(b2.shape),
            rep(w3.shape),
            rep(b3.shape),
        ],
        out_specs=pl.BlockSpec((blk, 1), lambda i: (i, 0)),
        out_shape=jax.ShapeDtypeStruct((BATCH, 1), jnp.float32),
    )(ue, me, w1u, w1m, b1, w2, b2, w3, b3)


def kernel(users, movies, user_table, movie_table, W1, b1, W2, b2, W3, b3):
    utab128 = user_table.reshape(-1, 128)
    mtab128 = movie_table.reshape(-1, 128)
    ue, me = _sc_gather(users.astype(jnp.int32), movies.astype(jnp.int32),
                        utab128, mtab128)
    w1t = W1.T
    w1u = w1t[:EMB]
    w1m = w1t[EMB:]
    return _tc_mlp(ue, me, w1u, w1m, b1.reshape(1, -1), W2.T,
                   b2.reshape(1, -1), W3.T, b3.reshape(1, 1))


# speedup vs baseline: 1.6656x; 1.6656x over previous
import functools

import jax
import jax.numpy as jnp
from jax import lax
from jax.experimental import pallas as pl
from jax.experimental.pallas import tpu as pltpu
from jax.experimental.pallas import tpu_sc as plsc

BATCH = 16384
EMB = 32

_info = plsc.get_sparse_core_info()
_NC, _NS = _info.num_cores, _info.num_subcores
_NW = _NC * _NS
_BPW = BATCH // _NW
_ST = 256


def _gather_body(users_hbm, movies_hbm, utab_hbm, mtab_hbm,
                 ue_hbm, me_hbm,
                 uidx_v, midx_v, ue_v, me_v, usem, msem):
    wid = lax.axis_index("s") * _NC + lax.axis_index("c")
    base = wid * _BPW
    pltpu.sync_copy(users_hbm.at[pl.ds(base, _BPW)], uidx_v)
    pltpu.sync_copy(movies_hbm.at[pl.ds(base, _BPW)], midx_v)

    iota = lax.iota(jnp.int32, 16)

    for h in range(_BPW // _ST):
        def _chunk(c, _):
            uv = uidx_v[pl.ds(h * _ST + c * 16, 16)]
            mv = midx_v[pl.ds(h * _ST + c * 16, 16)]
            for j in range(16):
                ur = lax.reduce_max(jnp.where(iota == j, uv, 0), axes=(0,))
                mr = lax.reduce_max(jnp.where(iota == j, mv, 0), axes=(0,))
                i = c * 16 + j
                pltpu.async_copy(utab_hbm.at[pl.ds(ur, 1)],
                                 ue_v.at[pl.ds(i, 1)], usem)
                pltpu.async_copy(mtab_hbm.at[pl.ds(mr, 1)],
                                 me_v.at[pl.ds(i, 1)], msem)
            return _
        lax.fori_loop(0, _ST // 16, _chunk, 0)

        pltpu.make_async_copy(ue_hbm.at[pl.ds(base, _ST)], ue_v, usem).wait()
        pltpu.make_async_copy(me_hbm.at[pl.ds(base, _ST)], me_v, msem).wait()
        pltpu.sync_copy(ue_v, ue_hbm.at[pl.ds(base + h * _ST, _ST)])
        pltpu.sync_copy(me_v, me_hbm.at[pl.ds(base + h * _ST, _ST)])


@jax.jit
def _sc_gather(users, movies, utab, mtab):
    mesh = plsc.VectorSubcoreMesh(core_axis_name="c", subcore_axis_name="s")
    fn = functools.partial(
        pl.kernel,
        mesh=mesh,
        compiler_params=pltpu.CompilerParams(needs_layout_passes=False),
        out_type=[
            jax.ShapeDtypeStruct((BATCH, EMB), jnp.float32),
            jax.ShapeDtypeStruct((BATCH, EMB), jnp.float32),
        ],
        scratch_types=[
            pltpu.VMEM((_BPW,), jnp.int32),
            pltpu.VMEM((_BPW,), jnp.int32),
            pltpu.VMEM((_ST, EMB), jnp.float32),
            pltpu.VMEM((_ST, EMB), jnp.float32),
            pltpu.SemaphoreType.DMA,
            pltpu.SemaphoreType.DMA,
        ],
    )(_gather_body)
    return fn(users, movies, utab, mtab)


def _mlp_body(ue_ref, me_ref, w1u_ref, w1m_ref, b1_ref, w2_ref, b2_ref,
              w3_ref, b3_ref, out_ref):
    h = jnp.dot(ue_ref[...], w1u_ref[...], preferred_element_type=jnp.float32)
    h += jnp.dot(me_ref[...], w1m_ref[...], preferred_element_type=jnp.float32)
    h = jax.nn.relu(h + b1_ref[...])
    h = jnp.dot(h, w2_ref[...], preferred_element_type=jnp.float32)
    h = jax.nn.relu(h + b2_ref[...])
    out = jnp.dot(h, w3_ref[...], preferred_element_type=jnp.float32)
    out_ref[...] = out + b3_ref[...]


@jax.jit
def _tc_mlp(ue, me, w1u, w1m, b1, w2, b2, w3, b3):
    blk = 2048
    grid = (BATCH // blk,)
    rep = lambda shape: pl.BlockSpec(shape, lambda i: (0, 0))
    return pl.pallas_call(
        _mlp_body,
        grid=grid,
        in_specs=[
            pl.BlockSpec((blk, EMB), lambda i: (i, 0)),
            pl.BlockSpec((blk, EMB), lambda i: (i, 0)),
            rep(w1u.shape),
            rep(w1m.shape),
            rep(b1.shape),
            rep(w2.shape),
            rep(b2.shape),
            rep(w3.shape),
            rep(b3.shape),
        ],
        out_specs=pl.BlockSpec((blk, 1), lambda i: (i, 0)),
        out_shape=jax.ShapeDtypeStruct((BATCH, 1), jnp.float32),
    )(ue, me, w1u, w1m, b1, w2, b2, w3, b3)


def kernel(users, movies, user_table, movie_table, W1, b1, W2, b2, W3, b3):
    ue, me = _sc_gather(users.astype(jnp.int32), movies.astype(jnp.int32),
                        user_table, movie_table)
    w1t = W1.T
    w1u = w1t[:EMB]
    w1m = w1t[EMB:]
    return _tc_mlp(ue, me, w1u, w1m, b1.reshape(1, -1), W2.T,
                   b2.reshape(1, -1), W3.T, b3.reshape(1, 1))


# speedup vs baseline: 1.8985x; 1.1399x over previous
import functools

import jax
import jax.numpy as jnp
from jax import lax
from jax.experimental import pallas as pl
from jax.experimental.pallas import tpu as pltpu
from jax.experimental.pallas import tpu_sc as plsc

BATCH = 16384
EMB = 32

_info = plsc.get_sparse_core_info()
_NC, _NS = _info.num_cores, _info.num_subcores
_NW = _NC * _NS
_BPW = BATCH // _NW
_ST = 256


def _transpose_body(inT_ref, out_ref):
    a = inT_ref[...]
    eye = (lax.broadcasted_iota(jnp.int32, (EMB, EMB), 0)
           == lax.broadcasted_iota(jnp.int32, (EMB, EMB), 1)).astype(jnp.float32)
    out_ref[...] = lax.dot_general(a, eye, (((0,), (0,)), ((), ())),
                                   preferred_element_type=jnp.float32)


@jax.jit
def _tc_transpose(tabT):
    n = tabT.shape[1]
    blk = 8192
    grid = (pl.cdiv(n, blk),)
    return pl.pallas_call(
        _transpose_body,
        grid=grid,
        in_specs=[pl.BlockSpec((EMB, blk), lambda i: (0, i))],
        out_specs=pl.BlockSpec((blk, EMB), lambda i: (i, 0)),
        out_shape=jax.ShapeDtypeStruct((n, EMB), jnp.float32),
    )(tabT)


def _gather_body(users_hbm, movies_hbm, utab_hbm, mtab_hbm,
                 ue_hbm, me_hbm,
                 uidx_v, midx_v, ue_v, me_v, usem, msem):
    wid = lax.axis_index("s") * _NC + lax.axis_index("c")
    base = wid * _BPW
    pltpu.sync_copy(users_hbm.at[pl.ds(base, _BPW)], uidx_v)
    pltpu.sync_copy(movies_hbm.at[pl.ds(base, _BPW)], midx_v)

    iota = lax.iota(jnp.int32, 16)

    for h in range(_BPW // _ST):
        def _chunk(c, _):
            uv = uidx_v[pl.ds(h * _ST + c * 16, 16)]
            mv = midx_v[pl.ds(h * _ST + c * 16, 16)]
            for j in range(16):
                ur = lax.reduce_max(jnp.where(iota == j, uv, 0), axes=(0,))
                mr = lax.reduce_max(jnp.where(iota == j, mv, 0), axes=(0,))
                i = c * 16 + j
                pltpu.async_copy(utab_hbm.at[pl.ds(ur, 1)],
                                 ue_v.at[pl.ds(i, 1)], usem)
                pltpu.async_copy(mtab_hbm.at[pl.ds(mr, 1)],
                                 me_v.at[pl.ds(i, 1)], msem)
            return _
        lax.fori_loop(0, _ST // 16, _chunk, 0)

        pltpu.make_async_copy(ue_hbm.at[pl.ds(base, _ST)], ue_v, usem).wait()
        pltpu.make_async_copy(me_hbm.at[pl.ds(base, _ST)], me_v, msem).wait()
        pltpu.sync_copy(ue_v, ue_hbm.at[pl.ds(base + h * _ST, _ST)])
        pltpu.sync_copy(me_v, me_hbm.at[pl.ds(base + h * _ST, _ST)])


@jax.jit
def _sc_gather(users, movies, utab, mtab):
    mesh = plsc.VectorSubcoreMesh(core_axis_name="c", subcore_axis_name="s")
    fn = functools.partial(
        pl.kernel,
        mesh=mesh,
        compiler_params=pltpu.CompilerParams(needs_layout_passes=False),
        out_type=[
            jax.ShapeDtypeStruct((BATCH, EMB), jnp.float32),
            jax.ShapeDtypeStruct((BATCH, EMB), jnp.float32),
        ],
        scratch_types=[
            pltpu.VMEM((_BPW,), jnp.int32),
            pltpu.VMEM((_BPW,), jnp.int32),
            pltpu.VMEM((_ST, EMB), jnp.float32),
            pltpu.VMEM((_ST, EMB), jnp.float32),
            pltpu.SemaphoreType.DMA,
            pltpu.SemaphoreType.DMA,
        ],
    )(_gather_body)
    return fn(users, movies, utab, mtab)


def _mlp_body(ue_ref, me_ref, w1u_ref, w1m_ref, b1_ref, w2_ref, b2_ref,
              w3_ref, b3_ref, out_ref):
    h = jnp.dot(ue_ref[...], w1u_ref[...], preferred_element_type=jnp.float32)
    h += jnp.dot(me_ref[...], w1m_ref[...], preferred_element_type=jnp.float32)
    h = jax.nn.relu(h + b1_ref[...])
    h = jnp.dot(h, w2_ref[...], preferred_element_type=jnp.float32)
    h = jax.nn.relu(h + b2_ref[...])
    out = jnp.dot(h, w3_ref[...], preferred_element_type=jnp.float32)
    out_ref[...] = out + b3_ref[...]


@jax.jit
def _tc_mlp(ue, me, w1u, w1m, b1, w2, b2, w3, b3):
    blk = 2048
    grid = (BATCH // blk,)
    rep = lambda shape: pl.BlockSpec(shape, lambda i: (0, 0))
    return pl.pallas_call(
        _mlp_body,
        grid=grid,
        in_specs=[
            pl.BlockSpec((blk, EMB), lambda i: (i, 0)),
            pl.BlockSpec((blk, EMB), lambda i: (i, 0)),
            rep(w1u.shape),
            rep(w1m.shape),
            rep(b1.shape),
            rep(w2.shape),
            rep(b2.shape),
            rep(w3.shape),
            rep(b3.shape),
        ],
        out_specs=pl.BlockSpec((blk, 1), lambda i: (i, 0)),
        out_shape=jax.ShapeDtypeStruct((BATCH, 1), jnp.float32),
    )(ue, me, w1u, w1m, b1, w2, b2, w3, b3)


def kernel(users, movies, user_table, movie_table, W1, b1, W2, b2, W3, b3):
    utab = _tc_transpose(user_table.T)
    mtab = _tc_transpose(movie_table.T)
    ue, me = _sc_gather(users.astype(jnp.int32), movies.astype(jnp.int32),
                        utab, mtab)
    w1t = W1.T
    w1u = w1t[:EMB]
    w1m = w1t[EMB:]
    return _tc_mlp(ue, me, w1u, w1m, b1.reshape(1, -1), W2.T,
                   b2.reshape(1, -1), W3.T, b3.reshape(1, 1))


# speedup vs baseline: 1.9384x; 1.0210x over previous
import functools

import jax
import jax.numpy as jnp
from jax import lax
from jax.experimental import pallas as pl
from jax.experimental.pallas import tpu as pltpu
from jax.experimental.pallas import tpu_sc as plsc

BATCH = 16384
EMB = 32

_info = plsc.get_sparse_core_info()
_NC, _NS = _info.num_cores, _info.num_subcores
_NW = _NC * _NS
_BPW = BATCH // _NW
_ST = 256


def _transpose_body(inT_ref, out_ref):
    out_ref[...] = inT_ref[...].T


@jax.jit
def _tc_transpose(tabT):
    n = tabT.shape[1]
    blk = 8192
    grid = (pl.cdiv(n, blk),)
    return pl.pallas_call(
        _transpose_body,
        grid=grid,
        in_specs=[pl.BlockSpec((EMB, blk), lambda i: (0, i))],
        out_specs=pl.BlockSpec((blk, EMB), lambda i: (i, 0)),
        out_shape=jax.ShapeDtypeStruct((n, EMB), jnp.float32),
    )(tabT)


def _gather_body(users_hbm, movies_hbm, utab_hbm, mtab_hbm,
                 ue_hbm, me_hbm,
                 uidx_v, midx_v, ue_v, me_v, usem, msem):
    wid = lax.axis_index("s") * _NC + lax.axis_index("c")
    base = wid * _BPW
    pltpu.sync_copy(users_hbm.at[pl.ds(base, _BPW)], uidx_v)
    pltpu.sync_copy(movies_hbm.at[pl.ds(base, _BPW)], midx_v)

    iota = lax.iota(jnp.int32, 16)

    for h in range(_BPW // _ST):
        def _chunk(c, _):
            uv = uidx_v[pl.ds(h * _ST + c * 16, 16)]
            mv = midx_v[pl.ds(h * _ST + c * 16, 16)]
            for j in range(16):
                ur = lax.reduce_max(jnp.where(iota == j, uv, 0), axes=(0,))
                mr = lax.reduce_max(jnp.where(iota == j, mv, 0), axes=(0,))
                i = c * 16 + j
                pltpu.async_copy(utab_hbm.at[pl.ds(ur, 1)],
                                 ue_v.at[pl.ds(i, 1)], usem)
                pltpu.async_copy(mtab_hbm.at[pl.ds(mr, 1)],
                                 me_v.at[pl.ds(i, 1)], msem)
            return _
        lax.fori_loop(0, _ST // 16, _chunk, 0)

        pltpu.make_async_copy(ue_hbm.at[pl.ds(base, _ST)], ue_v, usem).wait()
        pltpu.make_async_copy(me_hbm.at[pl.ds(base, _ST)], me_v, msem).wait()
        pltpu.sync_copy(ue_v, ue_hbm.at[pl.ds(base + h * _ST, _ST)])
        pltpu.sync_copy(me_v, me_hbm.at[pl.ds(base + h * _ST, _ST)])


@jax.jit
def _sc_gather(users, movies, utab, mtab):
    mesh = plsc.VectorSubcoreMesh(core_axis_name="c", subcore_axis_name="s")
    fn = functools.partial(
        pl.kernel,
        mesh=mesh,
        compiler_params=pltpu.CompilerParams(needs_layout_passes=False),
        out_type=[
            jax.ShapeDtypeStruct((BATCH, EMB), jnp.float32),
            jax.ShapeDtypeStruct((BATCH, EMB), jnp.float32),
        ],
        scratch_types=[
            pltpu.VMEM((_BPW,), jnp.int32),
            pltpu.VMEM((_BPW,), jnp.int32),
            pltpu.VMEM((_ST, EMB), jnp.float32),
            pltpu.VMEM((_ST, EMB), jnp.float32),
            pltpu.SemaphoreType.DMA,
            pltpu.SemaphoreType.DMA,
        ],
    )(_gather_body)
    return fn(users, movies, utab, mtab)


def _mlp_body(ue_ref, me_ref, w1u_ref, w1m_ref, b1_ref, w2_ref, b2_ref,
              w3_ref, b3_ref, out_ref):
    h = jnp.dot(ue_ref[...], w1u_ref[...], preferred_element_type=jnp.float32)
    h += jnp.dot(me_ref[...], w1m_ref[...], preferred_element_type=jnp.float32)
    h = jax.nn.relu(h + b1_ref[...])
    h = jnp.dot(h, w2_ref[...], preferred_element_type=jnp.float32)
    h = jax.nn.relu(h + b2_ref[...])
    out = jnp.dot(h, w3_ref[...], preferred_element_type=jnp.float32)
    out_ref[...] = out + b3_ref[...]


@jax.jit
def _tc_mlp(ue, me, w1u, w1m, b1, w2, b2, w3, b3):
    blk = 2048
    grid = (BATCH // blk,)
    rep = lambda shape: pl.BlockSpec(shape, lambda i: (0, 0))
    return pl.pallas_call(
        _mlp_body,
        grid=grid,
        in_specs=[
            pl.BlockSpec((blk, EMB), lambda i: (i, 0)),
            pl.BlockSpec((blk, EMB), lambda i: (i, 0)),
            rep(w1u.shape),
            rep(w1m.shape),
            rep(b1.shape),
            rep(w2.shape),
            rep(b2.shape),
            rep(w3.shape),
            rep(b3.shape),
        ],
        out_specs=pl.BlockSpec((blk, 1), lambda i: (i, 0)),
        out_shape=jax.ShapeDtypeStruct((BATCH, 1), jnp.float32),
    )(ue, me, w1u, w1m, b1, w2, b2, w3, b3)


def kernel(users, movies, user_table, movie_table, W1, b1, W2, b2, W3, b3):
    utab = _tc_transpose(user_table.T)
    mtab = _tc_transpose(movie_table.T)
    ue, me = _sc_gather(users.astype(jnp.int32), movies.astype(jnp.int32),
                        utab, mtab)
    w1t = W1.T
    w1u = w1t[:EMB]
    w1m = w1t[EMB:]
    return _tc_mlp(ue, me, w1u, w1m, b1.reshape(1, -1), W2.T,
                   b2.reshape(1, -1), W3.T, b3.reshape(1, 1))


# speedup vs baseline: 2.2057x; 1.1379x over previous
import functools

import jax
import jax.numpy as jnp
from jax import lax
from jax.experimental import pallas as pl
from jax.experimental.pallas import tpu as pltpu
from jax.experimental.pallas import tpu_sc as plsc

BATCH = 16384
EMB = 32

_info = plsc.get_sparse_core_info()
_NC, _NS = _info.num_cores, _info.num_subcores
_NW = _NC * _NS
_BPW = BATCH // _NW
_ST = 256


def _transpose_body(inT_ref, out_ref):
    out_ref[...] = inT_ref[...].T


@jax.jit
def _tc_transpose(tabT):
    n = tabT.shape[1]
    blk = 32768
    grid = (pl.cdiv(n, blk),)
    return pl.pallas_call(
        _transpose_body,
        grid=grid,
        in_specs=[pl.BlockSpec((EMB, blk), lambda i: (0, i))],
        out_specs=pl.BlockSpec((blk, EMB), lambda i: (i, 0)),
        out_shape=jax.ShapeDtypeStruct((n, EMB), jnp.float32),
    )(tabT)


def _gather_body(users_hbm, movies_hbm, utab_hbm, mtab_hbm,
                 ue_hbm, me_hbm,
                 uidx_v, midx_v, ue_v, me_v, usem, msem):
    wid = lax.axis_index("s") * _NC + lax.axis_index("c")
    base = wid * _BPW
    pltpu.sync_copy(users_hbm.at[pl.ds(base, _BPW)], uidx_v)
    pltpu.sync_copy(movies_hbm.at[pl.ds(base, _BPW)], midx_v)

    iota = lax.iota(jnp.int32, 16)

    for h in range(_BPW // _ST):
        def _chunk(c, _):
            uv = uidx_v[pl.ds(h * _ST + c * 16, 16)]
            mv = midx_v[pl.ds(h * _ST + c * 16, 16)]
            for j in range(16):
                ur = lax.reduce_max(jnp.where(iota == j, uv, 0), axes=(0,))
                mr = lax.reduce_max(jnp.where(iota == j, mv, 0), axes=(0,))
                i = c * 16 + j
                pltpu.async_copy(utab_hbm.at[pl.ds(ur, 1)],
                                 ue_v.at[pl.ds(i, 1)], usem)
                pltpu.async_copy(mtab_hbm.at[pl.ds(mr, 1)],
                                 me_v.at[pl.ds(i, 1)], msem)
            return _
        lax.fori_loop(0, _ST // 16, _chunk, 0)

        pltpu.make_async_copy(ue_hbm.at[pl.ds(base, _ST)], ue_v, usem).wait()
        pltpu.make_async_copy(me_hbm.at[pl.ds(base, _ST)], me_v, msem).wait()
        pltpu.sync_copy(ue_v, ue_hbm.at[pl.ds(base + h * _ST, _ST)])
        pltpu.sync_copy(me_v, me_hbm.at[pl.ds(base + h * _ST, _ST)])


@jax.jit
def _sc_gather(users, movies, utab, mtab):
    mesh = plsc.VectorSubcoreMesh(core_axis_name="c", subcore_axis_name="s")
    fn = functools.partial(
        pl.kernel,
        mesh=mesh,
        compiler_params=pltpu.CompilerParams(needs_layout_passes=False),
        out_type=[
            jax.ShapeDtypeStruct((BATCH, EMB), jnp.float32),
            jax.ShapeDtypeStruct((BATCH, EMB), jnp.float32),
        ],
        scratch_types=[
            pltpu.VMEM((_BPW,), jnp.int32),
            pltpu.VMEM((_BPW,), jnp.int32),
            pltpu.VMEM((_ST, EMB), jnp.float32),
            pltpu.VMEM((_ST, EMB), jnp.float32),
            pltpu.SemaphoreType.DMA,
            pltpu.SemaphoreType.DMA,
        ],
    )(_gather_body)
    return fn(users, movies, utab, mtab)


def _mlp_body(ue_ref, me_ref, w1u_ref, w1m_ref, b1_ref, w2_ref, b2_ref,
              w3_ref, b3_ref, out_ref):
    h = jnp.dot(ue_ref[...], w1u_ref[...], preferred_element_type=jnp.float32)
    h += jnp.dot(me_ref[...], w1m_ref[...], preferred_element_type=jnp.float32)
    h = jax.nn.relu(h + b1_ref[...])
    h = jnp.dot(h, w2_ref[...], preferred_element_type=jnp.float32)
    h = jax.nn.relu(h + b2_ref[...])
    out = jnp.dot(h, w3_ref[...], preferred_element_type=jnp.float32)
    out_ref[...] = out + b3_ref[...]


@jax.jit
def _tc_mlp(ue, me, w1u, w1m, b1, w2, b2, w3, b3):
    blk = 2048
    grid = (BATCH // blk,)
    rep = lambda shape: pl.BlockSpec(shape, lambda i: (0, 0))
    return pl.pallas_call(
        _mlp_body,
        grid=grid,
        in_specs=[
            pl.BlockSpec((blk, EMB), lambda i: (i, 0)),
            pl.BlockSpec((blk, EMB), lambda i: (i, 0)),
            rep(w1u.shape),
            rep(w1m.shape),
            rep(b1.shape),
            rep(w2.shape),
            rep(b2.shape),
            rep(w3.shape),
            rep(b3.shape),
        ],
        out_specs=pl.BlockSpec((blk, 1), lambda i: (i, 0)),
        out_shape=jax.ShapeDtypeStruct((BATCH, 1), jnp.float32),
    )(ue, me, w1u, w1m, b1, w2, b2, w3, b3)


def kernel(users, movies, user_table, movie_table, W1, b1, W2, b2, W3, b3):
    utab = _tc_transpose(user_table.T)
    mtab = _tc_transpose(movie_table.T)
    ue, me = _sc_gather(users.astype(jnp.int32), movies.astype(jnp.int32),
                        utab, mtab)
    w1t = W1.T
    w1u = w1t[:EMB]
    w1m = w1t[EMB:]
    return _tc_mlp(ue, me, w1u, w1m, b1.reshape(1, -1), W2.T,
                   b2.reshape(1, -1), W3.T, b3.reshape(1, 1))
